# baseline (speedup 1.0000x reference)
import functools

import jax
import jax.numpy as jnp
from jax import lax
from jax.experimental import pallas as pl
from jax.experimental.pallas import tpu as pltpu
from jax.experimental.pallas import tpu_sc as plsc

N = 10000
E = 320000
D = 128

NC = 2
NS = 16
NW = NC * NS
K = 80
NBT = 128
EP = NW * NBT * K
NBT0 = 248
NBT1 = 8
NBUF = 4
CH = 640
CHL = N - CH * (NS - 1)

_MESH = plsc.VectorSubcoreMesh(core_axis_name="c", subcore_axis_name="s")

_BCAST_DN = lax.GatherDimensionNumbers(
    offset_dims=(), collapsed_slice_dims=(0,), start_index_map=(0,))


def _bcast_lane(vec16, j):
    idx = jnp.full((16, 1), j, jnp.int32)
    return lax.gather(vec16, idx, _BCAST_DN, slice_sizes=(1,),
                      mode=lax.GatherScatterMode.PROMISE_IN_BOUNDS)


_DEG_SCRATCH = []
for _ in range(NBUF):
    _DEG_SCRATCH += [
        pltpu.VMEM((K,), jnp.int32),
        pltpu.VMEM((K,), jnp.float32),
        pltpu.SemaphoreType.DMA,
        pltpu.SemaphoreType.DMA,
    ]
_DEG_SCRATCH += [pltpu.VMEM_SHARED((N,), jnp.float32)]


@functools.partial(
    pl.kernel,
    out_type=jax.ShapeDtypeStruct((NC, N), jnp.float32),
    mesh=_MESH,
    scratch_types=_DEG_SCRATCH,
)
def _deg_kernel(dst_hbm, w_hbm, z_hbm, out_hbm, *scr):
    bufs = [scr[i * 4:(i + 1) * 4] for i in range(NBUF)]
    acc_s = scr[-1]
    c = lax.axis_index("c")
    s = lax.axis_index("s")

    @pl.when(s == 0)
    def _():
        pltpu.sync_copy(z_hbm, acc_s)

    plsc.subcore_barrier()

    base = (c * NS + s) * (NBT * K)

    def fetch(i, buf, wait_scatter):
        dst_v, w_v, lsem, ssem = buf
        if wait_scatter:
            pltpu.make_async_copy(w_v, acc_s.at[dst_v], ssem).wait()
        off = base + i * K
        pltpu.async_copy(dst_hbm.at[pl.ds(off, K)], dst_v, lsem)
        pltpu.async_copy(w_hbm.at[pl.ds(off, K)], w_v, lsem)

    def process(i, buf):
        dst_v, w_v, lsem, ssem = buf
        off = base + i * K
        pltpu.make_async_copy(dst_hbm.at[pl.ds(off, K)], dst_v, lsem).wait()
        pltpu.make_async_copy(w_hbm.at[pl.ds(off, K)], w_v, lsem).wait()
        pltpu.async_copy(w_v, acc_s.at[dst_v], ssem, add=True)

    fetch(0, bufs[0], False)
    fetch(1, bufs[1], False)

    def body(t, carry):
        for q in range(NBUF):
            process(t * NBUF + q, bufs[q])
            jb = bufs[(q + 2) % NBUF]
            if q < 2:
                @pl.when(t == 0)
                def _(jb=jb, j_off=q + 2):
                    fetch(j_off, jb, wait_scatter=False)

                @pl.when(t >= 1)
                def _(jb=jb, j_off=q + 2):
                    fetch(t * NBUF + j_off, jb, wait_scatter=True)
            else:
                @pl.when(t < NBT // NBUF - 1)
                def _(jb=jb, j_off=q + 2):
                    fetch(t * NBUF + j_off, jb, wait_scatter=True)
        return carry

    lax.fori_loop(0, NBT // NBUF, body, 0)

    for q in range(NBUF):
        dst_v, w_v, lsem, ssem = bufs[q]
        pltpu.make_async_copy(w_v, acc_s.at[dst_v], ssem).wait()

    plsc.subcore_barrier()

    @pl.when(s == 0)
    def _():
        pltpu.sync_copy(acc_s, out_hbm.at[c])


def _mm_scale(x, W, deg_cols):
    def body(x_ref, w_ref, d_ref, g_ref, dis_ref):
        h = jnp.dot(x_ref[...], w_ref[...], preferred_element_type=jnp.float32)
        deg = d_ref[:, 0:1] + d_ref[:, 1:2] + 1.0
        dis = lax.rsqrt(deg)
        dis_ref[...] = dis
        g_ref[...] = h * dis

    return pl.pallas_call(
        body,
        out_shape=(
            jax.ShapeDtypeStruct((N, D), jnp.float32),
            jax.ShapeDtypeStruct((N, 1), jnp.float32),
        ),
    )(x, W, deg_cols)


_AGG_SCRATCH = []
for _ in range(NBUF):
    _AGG_SCRATCH += [
        pltpu.VMEM((K,), jnp.int32),
        pltpu.VMEM((K,), jnp.int32),
        pltpu.VMEM((K,), jnp.float32),
        pltpu.VMEM((K, D), jnp.float32),
        pltpu.SemaphoreType.DMA,
        pltpu.SemaphoreType.DMA,
    ]
_AGG_SCRATCH += [pltpu.VMEM_SHARED((N, D), jnp.float32)]


@functools.partial(
    pl.kernel,
    out_type=jax.ShapeDtypeStruct((NC, N, D), jnp.float32),
    mesh=_MESH,
    scratch_types=_AGG_SCRATCH,
)
def _agg_kernel(g_hbm, src_hbm, dst_hbm, w_hbm, out_hbm, *scr):
    bufs = [scr[i * 6:(i + 1) * 6] for i in range(NBUF)]
    acc_s = scr[-1]
    c = lax.axis_index("c")
    s = lax.axis_index("s")

    row0 = pl.multiple_of(s * CH, 8)
    zrows = bufs[0][3]
    zv = jnp.zeros((16,), jnp.float32)

    def zbody(e, carry):
        for p in range(8):
            zrows[e, pl.ds(p * 16, 16)] = zv
        return carry

    lax.fori_loop(0, K, zbody, 0)

    @pl.when(s < NS - 1)
    def _():
        for kblk in range(CH // K):
            pltpu.sync_copy(zrows, acc_s.at[pl.ds(row0 + kblk * K, K)])

    @pl.when(s == NS - 1)
    def _():
        for kblk in range(CHL // K):
            pltpu.sync_copy(
                zrows, acc_s.at[pl.ds(CH * (NS - 1) + kblk * K, K)])

    plsc.subcore_barrier()

    base = jnp.where(c == 0, s * NBT0, NS * NBT0 + s * NBT1) * K
    trips = jnp.where(c == 0, NBT0 // NBUF, NBT1 // NBUF)

    def fetch(i, buf, wait_scatter):
        src_v, dst_v, w_v, rows_v, gsem, ssem = buf
        if wait_scatter:
            pltpu.make_async_copy(rows_v, acc_s.at[dst_v], ssem).wait()
        off = base + i * K
        pltpu.sync_copy(src_hbm.at[pl.ds(off, K)], src_v)
        pltpu.async_copy(dst_hbm.at[pl.ds(off, K)], dst_v, gsem)
        pltpu.async_copy(w_hbm.at[pl.ds(off, K)], w_v, gsem)
        pltpu.async_copy(g_hbm.at[src_v], rows_v, gsem)

    def process(i, buf):
        src_v, dst_v, w_v, rows_v, gsem, ssem = buf
        off = base + i * K
        pltpu.make_async_copy(dst_hbm.at[pl.ds(off, K)], dst_v, gsem).wait()
        pltpu.make_async_copy(w_hbm.at[pl.ds(off, K)], w_v, gsem).wait()
        pltpu.make_async_copy(g_hbm.at[src_v], rows_v, gsem).wait()

        def mulgrp(qq, carry2):
            wv = w_v[pl.ds(qq * 16, 16)]
            for j in range(16):
                wj = _bcast_lane(wv, j)
                e = qq * 16 + j
                for p in range(8):
                    rows_v[e, pl.ds(p * 16, 16)] = rows_v[e, pl.ds(p * 16, 16)] * wj
            return carry2

        lax.fori_loop(0, K // 16, mulgrp, 0)
        pltpu.async_copy(rows_v, acc_s.at[dst_v], ssem, add=True)

    fetch(0, bufs[0], False)
    fetch(1, bufs[1], False)

    def body(t, carry):
        for q in range(NBUF):
            process(t * NBUF + q, bufs[q])
            jb = bufs[(q + 2) % NBUF]
            if q < 2:
                @pl.when(t == 0)
                def _(jb=jb, j_off=q + 2):
                    fetch(j_off, jb, wait_scatter=False)

                @pl.when(t >= 1)
                def _(jb=jb, j_off=q + 2):
                    fetch(t * NBUF + j_off, jb, wait_scatter=True)
            else:
                @pl.when(t < trips - 1)
                def _(jb=jb, j_off=q + 2):
                    fetch(t * NBUF + j_off, jb, wait_scatter=True)
        return carry

    lax.fori_loop(0, trips, body, 0)

    for q in range(NBUF):
        src_v, dst_v, w_v, rows_v, gsem, ssem = bufs[q]
        pltpu.make_async_copy(rows_v, acc_s.at[dst_v], ssem).wait()

    plsc.subcore_barrier()

    @pl.when(s < NS - 1)
    def _():
        pltpu.sync_copy(acc_s.at[pl.ds(row0, CH)],
                        out_hbm.at[c, pl.ds(row0, CH)])

    @pl.when(s == NS - 1)
    def _():
        pltpu.sync_copy(acc_s.at[pl.ds(CH * (NS - 1), CHL)],
                        out_hbm.at[c, pl.ds(CH * (NS - 1), CHL)])


def _finish(accp, g, dis, bias_row):
    def body(a_ref, g_ref, dis_ref, b_ref, o_ref):
        acc = a_ref[0] + a_ref[1] + g_ref[...]
        o_ref[...] = jnp.maximum(dis_ref[...] * acc + b_ref[...], 0.0)

    return pl.pallas_call(
        body,
        out_shape=jax.ShapeDtypeStruct((N, D), jnp.float32),
    )(accp, g, dis, bias_row)


def kernel(x, edge_index, edge_weights, W, b):
    pad = EP - E
    src = jnp.concatenate([edge_index[0], jnp.zeros((pad,), jnp.int32)])
    dst = jnp.concatenate([edge_index[1], jnp.zeros((pad,), jnp.int32)])
    w = jnp.concatenate([edge_weights, jnp.zeros((pad,), jnp.float32)])
    z1 = jnp.zeros((N,), jnp.float32)

    degp = _deg_kernel(dst, w, z1)
    g, dis = _mm_scale(x, W, degp.T)
    accp = _agg_kernel(g, src, dst, w)
    return _finish(accp, g, dis, b.reshape(1, D))


# speedup vs baseline: 1.1065x; 1.1065x over previous
import functools

import jax
import jax.numpy as jnp
from jax import lax
from jax.experimental import pallas as pl
from jax.experimental.pallas import tpu as pltpu
from jax.experimental.pallas import tpu_sc as plsc

N = 10000
E = 320000
D = 128

NC = 2
NS = 16
NW = NC * NS
K = 80
NBT = 128
EP = NW * NBT * K
NBT0 = 240
NBT1 = 16
NBUF = 4
CH = 640
CHL = N - CH * (NS - 1)

_MESH = plsc.VectorSubcoreMesh(core_axis_name="c", subcore_axis_name="s")

_BCAST_DN = lax.GatherDimensionNumbers(
    offset_dims=(), collapsed_slice_dims=(0,), start_index_map=(0,))


def _bcast_lane(vec16, j):
    idx = jnp.full((16, 1), j, jnp.int32)
    return lax.gather(vec16, idx, _BCAST_DN, slice_sizes=(1,),
                      mode=lax.GatherScatterMode.PROMISE_IN_BOUNDS)


_DEG_SCRATCH = []
for _ in range(NBUF):
    _DEG_SCRATCH += [
        pltpu.VMEM((K,), jnp.int32),
        pltpu.VMEM((K,), jnp.float32),
        pltpu.SemaphoreType.DMA,
        pltpu.SemaphoreType.DMA,
    ]
_DEG_SCRATCH += [pltpu.VMEM_SHARED((N,), jnp.float32)]


@functools.partial(
    pl.kernel,
    out_type=jax.ShapeDtypeStruct((NC, N), jnp.float32),
    mesh=_MESH,
    scratch_types=_DEG_SCRATCH,
)
def _deg_kernel(dst_hbm, w_hbm, z_hbm, out_hbm, *scr):
    bufs = [scr[i * 4:(i + 1) * 4] for i in range(NBUF)]
    acc_s = scr[-1]
    c = lax.axis_index("c")
    s = lax.axis_index("s")

    @pl.when(s == 0)
    def _():
        pltpu.sync_copy(z_hbm, acc_s)

    plsc.subcore_barrier()

    base = (c * NS + s) * (NBT * K)

    def fetch(i, buf, wait_scatter):
        dst_v, w_v, lsem, ssem = buf
        if wait_scatter:
            pltpu.make_async_copy(w_v, acc_s.at[dst_v], ssem).wait()
        off = base + i * K
        pltpu.async_copy(dst_hbm.at[pl.ds(off, K)], dst_v, lsem)
        pltpu.async_copy(w_hbm.at[pl.ds(off, K)], w_v, lsem)

    def process(i, buf):
        dst_v, w_v, lsem, ssem = buf
        off = base + i * K
        pltpu.make_async_copy(dst_hbm.at[pl.ds(off, K)], dst_v, lsem).wait()
        pltpu.make_async_copy(w_hbm.at[pl.ds(off, K)], w_v, lsem).wait()
        pltpu.async_copy(w_v, acc_s.at[dst_v], ssem, add=True)

    fetch(0, bufs[0], False)
    fetch(1, bufs[1], False)

    def body(t, carry):
        for q in range(NBUF):
            process(t * NBUF + q, bufs[q])
            jb = bufs[(q + 2) % NBUF]
            if q < 2:
                @pl.when(t == 0)
                def _(jb=jb, j_off=q + 2):
                    fetch(j_off, jb, wait_scatter=False)

                @pl.when(t >= 1)
                def _(jb=jb, j_off=q + 2):
                    fetch(t * NBUF + j_off, jb, wait_scatter=True)
            else:
                @pl.when(t < NBT // NBUF - 1)
                def _(jb=jb, j_off=q + 2):
                    fetch(t * NBUF + j_off, jb, wait_scatter=True)
        return carry

    lax.fori_loop(0, NBT // NBUF, body, 0)

    for q in range(NBUF):
        dst_v, w_v, lsem, ssem = bufs[q]
        pltpu.make_async_copy(w_v, acc_s.at[dst_v], ssem).wait()

    plsc.subcore_barrier()

    @pl.when(s == 0)
    def _():
        pltpu.sync_copy(acc_s, out_hbm.at[c])


def _mm_scale(x, W, deg_cols):
    def body(x_ref, w_ref, d_ref, g_ref, dis_ref):
        h = jnp.dot(x_ref[...], w_ref[...], preferred_element_type=jnp.float32)
        deg = d_ref[:, 0:1] + d_ref[:, 1:2] + 1.0
        dis = lax.rsqrt(deg)
        dis_ref[...] = dis
        g_ref[...] = h * dis

    return pl.pallas_call(
        body,
        out_shape=(
            jax.ShapeDtypeStruct((N, D), jnp.float32),
            jax.ShapeDtypeStruct((N, 1), jnp.float32),
        ),
    )(x, W, deg_cols)


_AGG_SCRATCH = []
for _ in range(NBUF):
    _AGG_SCRATCH += [
        pltpu.VMEM((K,), jnp.int32),
        pltpu.VMEM((K,), jnp.int32),
        pltpu.VMEM((K,), jnp.float32),
        pltpu.VMEM((K, D), jnp.float32),
        pltpu.SemaphoreType.DMA,
        pltpu.SemaphoreType.DMA,
    ]
_AGG_SCRATCH += [pltpu.VMEM_SHARED((N, D), jnp.float32)]


@functools.partial(
    pl.kernel,
    out_type=jax.ShapeDtypeStruct((NC, N, D), jnp.float32),
    mesh=_MESH,
    scratch_types=_AGG_SCRATCH,
)
def _agg_kernel(g_hbm, src_hbm, dst_hbm, w_hbm, out_hbm, *scr):
    bufs = [scr[i * 6:(i + 1) * 6] for i in range(NBUF)]
    acc_s = scr[-1]
    c = lax.axis_index("c")
    s = lax.axis_index("s")

    row0 = pl.multiple_of(s * CH, 8)
    zrows = bufs[0][3]
    zv = jnp.zeros((16,), jnp.float32)

    def zbody(e, carry):
        for p in range(8):
            zrows[e, pl.ds(p * 16, 16)] = zv
        return carry

    lax.fori_loop(0, K, zbody, 0)

    @pl.when(s < NS - 1)
    def _():
        for kblk in range(CH // K):
            pltpu.sync_copy(zrows, acc_s.at[pl.ds(row0 + kblk * K, K)])

    @pl.when(s == NS - 1)
    def _():
        for kblk in range(CHL // K):
            pltpu.sync_copy(
                zrows, acc_s.at[pl.ds(CH * (NS - 1) + kblk * K, K)])

    plsc.subcore_barrier()

    base = jnp.where(c == 0, s * NBT0, NS * NBT0 + s * NBT1) * K
    trips = jnp.where(c == 0, NBT0 // NBUF, NBT1 // NBUF)

    def fetch(i, buf, wait_scatter):
        src_v, dst_v, w_v, rows_v, gsem, ssem = buf
        if wait_scatter:
            pltpu.make_async_copy(rows_v, acc_s.at[dst_v], ssem).wait()
        off = base + i * K
        pltpu.sync_copy(src_hbm.at[pl.ds(off, K)], src_v)
        pltpu.async_copy(dst_hbm.at[pl.ds(off, K)], dst_v, gsem)
        pltpu.async_copy(w_hbm.at[pl.ds(off, K)], w_v, gsem)
        pltpu.async_copy(g_hbm.at[src_v], rows_v, gsem)

    def process(i, buf):
        src_v, dst_v, w_v, rows_v, gsem, ssem = buf
        off = base + i * K
        pltpu.make_async_copy(dst_hbm.at[pl.ds(off, K)], dst_v, gsem).wait()
        pltpu.make_async_copy(w_hbm.at[pl.ds(off, K)], w_v, gsem).wait()
        pltpu.make_async_copy(g_hbm.at[src_v], rows_v, gsem).wait()

        def mulgrp(qq, carry2):
            wv = w_v[pl.ds(qq * 16, 16)]
            for j in range(16):
                wj = _bcast_lane(wv, j)
                e = qq * 16 + j
                for p in range(8):
                    rows_v[e, pl.ds(p * 16, 16)] = rows_v[e, pl.ds(p * 16, 16)] * wj
            return carry2

        lax.fori_loop(0, K // 16, mulgrp, 0)
        pltpu.async_copy(rows_v, acc_s.at[dst_v], ssem, add=True)

    fetch(0, bufs[0], False)
    fetch(1, bufs[1], False)

    def body(t, carry):
        for q in range(NBUF):
            process(t * NBUF + q, bufs[q])
            jb = bufs[(q + 2) % NBUF]
            if q < 2:
                @pl.when(t == 0)
                def _(jb=jb, j_off=q + 2):
                    fetch(j_off, jb, wait_scatter=False)

                @pl.when(t >= 1)
                def _(jb=jb, j_off=q + 2):
                    fetch(t * NBUF + j_off, jb, wait_scatter=True)
            else:
                @pl.when(t < trips - 1)
                def _(jb=jb, j_off=q + 2):
                    fetch(t * NBUF + j_off, jb, wait_scatter=True)
        return carry

    lax.fori_loop(0, trips, body, 0)

    for q in range(NBUF):
        src_v, dst_v, w_v, rows_v, gsem, ssem = bufs[q]
        pltpu.make_async_copy(rows_v, acc_s.at[dst_v], ssem).wait()

    plsc.subcore_barrier()

    @pl.when(s < NS - 1)
    def _():
        pltpu.sync_copy(acc_s.at[pl.ds(row0, CH)],
                        out_hbm.at[c, pl.ds(row0, CH)])

    @pl.when(s == NS - 1)
    def _():
        pltpu.sync_copy(acc_s.at[pl.ds(CH * (NS - 1), CHL)],
                        out_hbm.at[c, pl.ds(CH * (NS - 1), CHL)])


def _finish(accp, g, dis, bias_row):
    def body(a_ref, g_ref, dis_ref, b_ref, o_ref):
        acc = a_ref[0] + a_ref[1] + g_ref[...]
        o_ref[...] = jnp.maximum(dis_ref[...] * acc + b_ref[...], 0.0)

    return pl.pallas_call(
        body,
        out_shape=jax.ShapeDtypeStruct((N, D), jnp.float32),
    )(accp, g, dis, bias_row)


def kernel(x, edge_index, edge_weights, W, b):
    pad = EP - E
    src = jnp.concatenate([edge_index[0], jnp.zeros((pad,), jnp.int32)])
    dst = jnp.concatenate([edge_index[1], jnp.zeros((pad,), jnp.int32)])
    w = jnp.concatenate([edge_weights, jnp.zeros((pad,), jnp.float32)])
    z1 = jnp.zeros((N,), jnp.float32)

    degp = _deg_kernel(dst, w, z1)
    g, dis = _mm_scale(x, W, degp.T)
    accp = _agg_kernel(g, src, dst, w)
    return _finish(accp, g, dis, b.reshape(1, D))
